# baseline (speedup 1.0000x reference)
import functools

import jax
import jax.numpy as jnp
from jax import lax
from jax.experimental import pallas as pl
from jax.experimental.pallas import tpu as pltpu
from jax.experimental.pallas import tpu_sc as plsc

HIDDEN = 64
CHUNKS = HIDDEN // 16


def _make_sc_kernel(batch):
    info = plsc.get_sparse_core_info()
    nc, ns = info.num_cores, info.num_subcores
    nw = nc * ns
    assert batch % (8 * nw) == 0
    bpw = batch // nw

    mesh = plsc.VectorSubcoreMesh(core_axis_name="c", subcore_axis_name="s")

    @functools.partial(
        pl.kernel,
        mesh=mesh,
        compiler_params=pltpu.CompilerParams(
            needs_layout_passes=False, use_tc_tiling_on_sc=False
        ),
        out_type=jax.ShapeDtypeStruct((batch,), jnp.float32),
        scratch_types=[
            pltpu.VMEM((bpw,), jnp.int32),
            pltpu.VMEM((bpw,), jnp.int32),
            pltpu.VMEM((bpw,), jnp.int32),
            pltpu.VMEM((bpw, HIDDEN), jnp.float32),
            pltpu.VMEM((bpw, HIDDEN), jnp.float32),
            pltpu.VMEM((bpw, HIDDEN), jnp.float32),
            pltpu.VMEM((16 * 16,), jnp.float32),
            pltpu.VMEM((bpw,), jnp.float32),
            pltpu.SemaphoreType.DMA,
        ],
    )
    def k(ent_hbm, rel_hbm, hidx_hbm, ridx_hbm, tidx_hbm, out_hbm,
          hidx_v, ridx_v, tidx_v, hrow_v, rrow_v, trow_v, tmp_v, score_v,
          sem):
        wid = lax.axis_index("s") * nc + lax.axis_index("c")
        base = wid * bpw
        pltpu.sync_copy(hidx_hbm.at[pl.ds(base, bpw)], hidx_v)
        pltpu.sync_copy(ridx_hbm.at[pl.ds(base, bpw)], ridx_v)
        pltpu.sync_copy(tidx_hbm.at[pl.ds(base, bpw)], tidx_v)

        c1 = pltpu.async_copy(ent_hbm.at[hidx_v], hrow_v, sem)
        c2 = pltpu.async_copy(rel_hbm.at[ridx_v], rrow_v, sem)
        c3 = pltpu.async_copy(ent_hbm.at[tidx_v], trow_v, sem)
        c1.wait()
        c2.wait()
        c3.wait()

        lanes16 = lax.iota(jnp.int32, 16) * 16

        def group(g, carry):
            for j in range(16):
                s = g * 16 + j
                acc = (hrow_v[s, pl.ds(0, 16)]
                       * rrow_v[s, pl.ds(0, 16)]
                       * trow_v[s, pl.ds(0, 16)])
                for c in range(1, CHUNKS):
                    o = c * 16
                    acc = acc + (hrow_v[s, pl.ds(o, 16)]
                                 * rrow_v[s, pl.ds(o, 16)]
                                 * trow_v[s, pl.ds(o, 16)])
                tmp_v[pl.ds(j * 16, 16)] = acc
            tot = plsc.load_gather(tmp_v, [lanes16])
            for d in range(1, 16):
                tot = tot + plsc.load_gather(tmp_v, [lanes16 + d])
            score_v[pl.ds(g * 16, 16)] = tot
            return carry

        lax.fori_loop(0, bpw // 16, group, 0)

        pltpu.sync_copy(score_v, out_hbm.at[pl.ds(base, bpw)])

    return k


@jax.jit
def kernel(entity_embedding, relation_embedding, sample):
    batch = sample.shape[0]
    hidx = sample[:, 0]
    ridx = sample[:, 1]
    tidx = sample[:, 2]
    k = _make_sc_kernel(batch)
    score = k(entity_embedding, relation_embedding, hidx, ridx, tidx)
    return score.reshape(batch, 1)
